# baseline (speedup 1.0000x reference)
import jax
import jax.numpy as jnp
from jax import lax
from jax.experimental import pallas as pl
from jax.experimental.pallas import tpu as pltpu
from jax.experimental.pallas import tpu_sc as plsc

K = 11
MAXLEN = 4000
NSUB = 600
NBBOX = 8
NKP = 16
B = 1024
NTOT = MAXLEN + NSUB + 2 * NBBOX + 2 * NKP
VB = 16
NPAD = 4736
VPB = NPAD // VB
NPLANE = 5
BPH = B // 2
NSTRIP = VPB // 8
NG = BPH // 16


def _coef_body(ang_ref, gx_ref, gy_ref, gz_ref, tx_ref, ty_ref, tz_ref, out_ref):
    a = ang_ref[...]
    gx = gx_ref[...]
    gy = gy_ref[...]
    gz = gz_ref[...]

    n2 = gx * gx + gy * gy + gz * gz
    n = jnp.sqrt(n2)
    half = 0.5 * n
    small = n < 1e-6
    safe = jnp.where(small, 1.0, n)
    kf = jnp.where(small, 0.5 - n2 / 48.0, jnp.sin(half) / safe)
    w = jnp.cos(half)
    x = gx * kf
    y = gy * kf
    z = gz * kf
    wu = w * w - (x * x + y * y + z * z)
    out_ref[0] = wu + 2.0 * x * x
    out_ref[1] = 2.0 * (x * y - w * z)
    out_ref[2] = 2.0 * (x * z + w * y)
    out_ref[3] = 2.0 * (x * y + w * z)
    out_ref[4] = wu + 2.0 * y * y
    out_ref[5] = 2.0 * (y * z - w * x)
    out_ref[6] = 2.0 * (x * z - w * y)
    out_ref[7] = 2.0 * (y * z + w * x)
    out_ref[8] = wu + 2.0 * z * z

    an = jnp.abs(a)
    ah = 0.5 * an
    asmall = an < 1e-6
    asafe = jnp.where(asmall, 1.0, an)
    ak = jnp.where(asmall, 0.5 - an * an / 48.0, jnp.sin(ah) / asafe)
    aw = jnp.cos(ah)
    az = -a * ak
    out_ref[9] = aw * aw - az * az
    out_ref[10] = 2.0 * aw * az
    out_ref[11] = aw * aw + az * az
    out_ref[12] = tx_ref[...]
    out_ref[13] = ty_ref[...]
    out_ref[14] = tz_ref[...]
    out_ref[15] = jnp.zeros_like(a)


def _compute_coefs(angles, global_orient, transl):
    ang = angles.reshape(8, 128)
    g = global_orient.T.reshape(3, 8, 128)
    t = transl.T.reshape(3, 8, 128)
    planes = pl.pallas_call(
        _coef_body,
        out_shape=jax.ShapeDtypeStruct((16, 8, 128), jnp.float32),
    )(ang, g[0], g[1], g[2], t[0], t[1], t[2])
    return planes.reshape(16 * B)


def _sc_body(tabf_hbm, cT_hbm, idx_hbm, diam_hbm,
             vT_hbm, sT_hbm, bT_hbm, kT_hbm, mT_hbm, dout_hbm, scrT_hbm,
             tab_v, ctab_v, ktab_v, dtab_v, dbuf_v, sbufA_v, sbufB_v,
             sem, semA, semB):
    vb = lax.axis_index("s")
    bh = lax.axis_index("c")
    voff = pl.multiple_of(vb * VPB, 8)
    boff = pl.multiple_of(bh * BPH, BPH)

    copies = []
    for k in range(K):
        for c in range(NPLANE):
            so = (k * NPLANE + c) * NPAD
            copies.append(pltpu.async_copy(
                tabf_hbm.at[pl.ds(pl.multiple_of(so + voff, 8), VPB)],
                tab_v.at[pl.ds((k * NPLANE + c) * VPB, VPB)], sem))
    for j in range(16):
        copies.append(pltpu.async_copy(
            cT_hbm.at[pl.ds(pl.multiple_of(j * B + boff, 8), BPH)],
            ctab_v.at[pl.ds(j * BPH, BPH)], sem))
    copies.append(pltpu.async_copy(idx_hbm.at[pl.ds(boff, BPH)], ktab_v, sem))
    copies.append(pltpu.async_copy(diam_hbm, dtab_v, sem))
    for cp in copies:
        cp.wait()

    @pl.when(vb == 0)
    def _():
        def diam_g(g, _):
            kvec = ktab_v[pl.ds(16 * g, 16)]
            dbuf_v[pl.ds(16 * g, 16)] = plsc.load_gather(dtab_v, [kvec])
            return 0
        lax.fori_loop(0, NG, diam_g, 0)
        pltpu.sync_copy(dbuf_v, dout_hbm.at[pl.ds(boff, BPH)])

    def compute_strip(s, sbuf):
        @plsc.parallel_loop(0, NG, step=1, unroll=2)
        def per_group(g):
            kvec = ktab_v[pl.ds(16 * g, 16)]
            kb = kvec * (NPLANE * VPB)
            cv = [ctab_v[pl.ds(j * BPH + 16 * g, 16)] for j in range(15)]
            (m00, m01, m02, m10, m11, m12, m20, m21, m22,
             cC, cS, cZ, tx, ty, tz) = cv
            for nl in range(8):
                nidx = 8 * s + nl
                xv = plsc.load_gather(tab_v, [kb + nidx])
                yv = plsc.load_gather(tab_v, [kb + (VPB + nidx)])
                zv = plsc.load_gather(tab_v, [kb + (2 * VPB + nidx)])
                fv = plsc.load_gather(tab_v, [kb + (3 * VPB + nidx)])
                mv = plsc.load_gather(tab_v, [kb + (4 * VPB + nidx)])
                m = fv > 0.5
                xe = jnp.where(m, cC * xv - cS * yv, xv)
                ye = jnp.where(m, cS * xv + cC * yv, yv)
                ze = jnp.where(m, cZ * zv, zv)
                ox = m00 * xe + m01 * ye + m02 * ze + tx
                oy = m10 * xe + m11 * ye + m12 * ze + ty
                oz = m20 * xe + m21 * ye + m22 * ze + tz
                sbuf[nl, pl.ds(16 * g, 16)] = ox
                sbuf[8 + nl, pl.ds(16 * g, 16)] = oy
                sbuf[16 + nl, pl.ds(16 * g, 16)] = oz
                sbuf[24 + nl, pl.ds(16 * g, 16)] = mv

    def fire_strip(s, sbuf, sem):
        gn0 = pl.multiple_of(voff + 8 * s, 8)

        def row(q):
            return sbuf.at[pl.ds(8 * q, 8), pl.ds(0, BPH)]

        def scr(q):
            return scrT_hbm.at[pl.ds(0, 8), pl.ds(boff, BPH)]

        def out3(dst_hbm, loc, last):
            loc = pl.multiple_of(loc, 8)
            for c in range(3):
                pltpu.async_copy(
                    row(c), dst_hbm.at[c, pl.ds(loc, 8), pl.ds(boff, BPH)],
                    sem)
            pltpu.async_copy(row(3), last, sem)

        def v_case():
            out3(vT_hbm, gn0,
                 mT_hbm.at[pl.ds(pl.multiple_of(gn0, 8), 8), pl.ds(boff, BPH)])

        def rest1():
            lax.cond(gn0 < MAXLEN + NSUB,
                     lambda: out3(sT_hbm, gn0 - MAXLEN, scr(3)), rest2)

        def rest2():
            lax.cond(gn0 < MAXLEN + NSUB + 2 * NBBOX,
                     lambda: out3(bT_hbm, gn0 - (MAXLEN + NSUB), scr(3)),
                     rest3)

        def rest3():
            def k_case():
                out3(kT_hbm, gn0 - (MAXLEN + NSUB + 2 * NBBOX), scr(3))

            def pad_case():
                for q in range(4):
                    pltpu.async_copy(row(q), scr(q), sem)

            lax.cond(gn0 < NTOT, k_case, pad_case)

        lax.cond(gn0 < MAXLEN, v_case, rest1)

    def drain4(sbuf, sem):
        for q in range(4):
            pltpu.make_async_copy(
                scrT_hbm.at[pl.ds(0, 8), pl.ds(boff, BPH)],
                sbuf.at[pl.ds(8 * q, 8), pl.ds(0, BPH)], sem).wait()

    def pair_body(i, _):
        @pl.when(i > 0)
        def _():
            drain4(sbufB_v, semB)
        compute_strip(2 * i, sbufA_v)
        fire_strip(2 * i, sbufA_v, semA)
        compute_strip(2 * i + 1, sbufB_v)
        drain4(sbufA_v, semA)
        fire_strip(2 * i + 1, sbufB_v, semB)
        return 0

    lax.fori_loop(0, NSTRIP // 2, pair_body, 0)
    drain4(sbufB_v, semB)
    compute_strip(NSTRIP - 1, sbufA_v)
    fire_strip(NSTRIP - 1, sbufA_v, semA)
    drain4(sbufA_v, semA)


_sc_kernel = pl.kernel(
    _sc_body,
    out_type=(
        jax.ShapeDtypeStruct((3, MAXLEN, B), jnp.float32),
        jax.ShapeDtypeStruct((3, NSUB, B), jnp.float32),
        jax.ShapeDtypeStruct((3, 2 * NBBOX, B), jnp.float32),
        jax.ShapeDtypeStruct((3, 2 * NKP, B), jnp.float32),
        jax.ShapeDtypeStruct((MAXLEN, B), jnp.float32),
        jax.ShapeDtypeStruct((B,), jnp.float32),
        jax.ShapeDtypeStruct((8, B), jnp.float32),
    ),
    mesh=plsc.VectorSubcoreMesh(core_axis_name="c", subcore_axis_name="s"),
    compiler_params=pltpu.CompilerParams(needs_layout_passes=False),
    scratch_types=[
        pltpu.VMEM((K * NPLANE * VPB,), jnp.float32),
        pltpu.VMEM((16 * BPH,), jnp.float32),
        pltpu.VMEM((BPH,), jnp.int32),
        pltpu.VMEM((16,), jnp.float32),
        pltpu.VMEM((BPH,), jnp.float32),
        pltpu.VMEM((32, BPH), jnp.float32),
        pltpu.VMEM((32, BPH), jnp.float32),
        pltpu.SemaphoreType.DMA,
        pltpu.SemaphoreType.DMA,
        pltpu.SemaphoreType.DMA,
    ],
)


def kernel(angles, global_orient, transl, obj_idx, v_tab, v_sub_tab, parts_tab,
           parts_sub_tab, bbox_top_tab, bbox_bottom_tab, kp_top_tab,
           kp_bottom_tab, mask_tab, diameter_tab):
    obj_idx = obj_idx.astype(jnp.int32)

    coords = jnp.concatenate(
        [v_tab, v_sub_tab, bbox_top_tab, bbox_bottom_tab, kp_top_tab,
         kp_bottom_tab], axis=1)
    flags = jnp.concatenate(
        [(parts_tab == 1).astype(jnp.float32),
         (parts_sub_tab == 1).astype(jnp.float32),
         jnp.ones((K, NBBOX), jnp.float32),
         jnp.zeros((K, NBBOX), jnp.float32),
         jnp.ones((K, NKP), jnp.float32),
         jnp.zeros((K, NKP), jnp.float32)], axis=1)
    maskp = jnp.pad(mask_tab.astype(jnp.float32), ((0, 0), (0, NTOT - MAXLEN)))
    planar = jnp.concatenate(
        [jnp.transpose(coords, (0, 2, 1)), flags[:, None, :],
         maskp[:, None, :]], axis=1)
    tabf = jnp.pad(planar, ((0, 0), (0, 0), (0, NPAD - NTOT))).reshape(-1)

    diam_pad = jnp.pad(diameter_tab.astype(jnp.float32), (0, 16 - K))
    coefT = _compute_coefs(angles, global_orient, transl)

    vT, sT, bT, kT, mT, diameter, _scr = _sc_kernel(
        tabf, coefT, obj_idx, diam_pad)
    return (jnp.transpose(vT, (2, 1, 0)), jnp.transpose(sT, (2, 1, 0)),
            jnp.transpose(bT, (2, 1, 0)), jnp.transpose(kT, (2, 1, 0)),
            diameter, jnp.transpose(mT, (1, 0)))


# speedup vs baseline: 1.0679x; 1.0679x over previous
import jax
import jax.numpy as jnp
from jax import lax
from jax.experimental import pallas as pl
from jax.experimental.pallas import tpu as pltpu
from jax.experimental.pallas import tpu_sc as plsc

K = 11
MAXLEN = 4000
NSUB = 600
NBBOX = 8
NKP = 16
B = 1024
NTOT = MAXLEN + NSUB + 2 * NBBOX + 2 * NKP
VB = 16
NPAD = 4736
VPB = NPAD // VB
NPLANE = 5
BPH = B // 2
NSTRIP = VPB // 8
NG = BPH // 16


def _coef_body(ang_ref, gx_ref, gy_ref, gz_ref, tx_ref, ty_ref, tz_ref, out_ref):
    a = ang_ref[...]
    gx = gx_ref[...]
    gy = gy_ref[...]
    gz = gz_ref[...]

    n2 = gx * gx + gy * gy + gz * gz
    n = jnp.sqrt(n2)
    half = 0.5 * n
    small = n < 1e-6
    safe = jnp.where(small, 1.0, n)
    kf = jnp.where(small, 0.5 - n2 / 48.0, jnp.sin(half) / safe)
    w = jnp.cos(half)
    x = gx * kf
    y = gy * kf
    z = gz * kf
    wu = w * w - (x * x + y * y + z * z)
    out_ref[0] = wu + 2.0 * x * x
    out_ref[1] = 2.0 * (x * y - w * z)
    out_ref[2] = 2.0 * (x * z + w * y)
    out_ref[3] = 2.0 * (x * y + w * z)
    out_ref[4] = wu + 2.0 * y * y
    out_ref[5] = 2.0 * (y * z - w * x)
    out_ref[6] = 2.0 * (x * z - w * y)
    out_ref[7] = 2.0 * (y * z + w * x)
    out_ref[8] = wu + 2.0 * z * z

    an = jnp.abs(a)
    ah = 0.5 * an
    asmall = an < 1e-6
    asafe = jnp.where(asmall, 1.0, an)
    ak = jnp.where(asmall, 0.5 - an * an / 48.0, jnp.sin(ah) / asafe)
    aw = jnp.cos(ah)
    az = -a * ak
    out_ref[9] = aw * aw - az * az
    out_ref[10] = 2.0 * aw * az
    out_ref[11] = aw * aw + az * az
    out_ref[12] = tx_ref[...]
    out_ref[13] = ty_ref[...]
    out_ref[14] = tz_ref[...]
    out_ref[15] = jnp.zeros_like(a)


def _compute_coefs(angles, global_orient, transl):
    ang = angles.reshape(8, 128)
    g = global_orient.T.reshape(3, 8, 128)
    t = transl.T.reshape(3, 8, 128)
    planes = pl.pallas_call(
        _coef_body,
        out_shape=jax.ShapeDtypeStruct((16, 8, 128), jnp.float32),
    )(ang, g[0], g[1], g[2], t[0], t[1], t[2])
    return planes.reshape(16 * B)


def _sc_body(tabf_hbm, cT_hbm, idx_hbm, diam_hbm,
             vT_hbm, sT_hbm, bT_hbm, kT_hbm, mT_hbm, dout_hbm, scrT_hbm,
             tab_v, ctab_v, ktab_v, dtab_v, dbuf_v, sbufA_v, sbufB_v,
             sbufC_v, sbufD_v, sem, semA, semB, semC, semD):
    vb = lax.axis_index("s")
    bh = lax.axis_index("c")
    voff = pl.multiple_of(vb * VPB, 8)
    boff = pl.multiple_of(bh * BPH, BPH)

    copies = []
    for k in range(K):
        for c in range(NPLANE):
            so = (k * NPLANE + c) * NPAD
            copies.append(pltpu.async_copy(
                tabf_hbm.at[pl.ds(pl.multiple_of(so + voff, 8), VPB)],
                tab_v.at[pl.ds((k * NPLANE + c) * VPB, VPB)], sem))
    for j in range(16):
        copies.append(pltpu.async_copy(
            cT_hbm.at[pl.ds(pl.multiple_of(j * B + boff, 8), BPH)],
            ctab_v.at[pl.ds(j * BPH, BPH)], sem))
    copies.append(pltpu.async_copy(idx_hbm.at[pl.ds(boff, BPH)], ktab_v, sem))
    copies.append(pltpu.async_copy(diam_hbm, dtab_v, sem))
    for cp in copies:
        cp.wait()

    @pl.when(vb == 0)
    def _():
        def diam_g(g, _):
            kvec = ktab_v[pl.ds(16 * g, 16)]
            dbuf_v[pl.ds(16 * g, 16)] = plsc.load_gather(dtab_v, [kvec])
            return 0
        lax.fori_loop(0, NG, diam_g, 0)
        pltpu.sync_copy(dbuf_v, dout_hbm.at[pl.ds(boff, BPH)])

    def _compute(s, bufs, nrows):
        def per_group(g, _):
            kvec = ktab_v[pl.ds(16 * g, 16)]
            kb = kvec * (NPLANE * VPB)
            cv = [ctab_v[pl.ds(j * BPH + 16 * g, 16)] for j in range(15)]
            (m00, m01, m02, m10, m11, m12, m20, m21, m22,
             cC, cS, cZ, tx, ty, tz) = cv
            for nl in range(nrows):
                sbuf = bufs[nl // 8]
                r = nl % 8
                nidx = 8 * s + nl
                xv = plsc.load_gather(tab_v, [kb + nidx])
                yv = plsc.load_gather(tab_v, [kb + (VPB + nidx)])
                zv = plsc.load_gather(tab_v, [kb + (2 * VPB + nidx)])
                fv = plsc.load_gather(tab_v, [kb + (3 * VPB + nidx)])
                mv = plsc.load_gather(tab_v, [kb + (4 * VPB + nidx)])
                m = fv > 0.5
                xe = jnp.where(m, cC * xv - cS * yv, xv)
                ye = jnp.where(m, cS * xv + cC * yv, yv)
                ze = jnp.where(m, cZ * zv, zv)
                ox = m00 * xe + m01 * ye + m02 * ze + tx
                oy = m10 * xe + m11 * ye + m12 * ze + ty
                oz = m20 * xe + m21 * ye + m22 * ze + tz
                sbuf[r, pl.ds(16 * g, 16)] = ox
                sbuf[8 + r, pl.ds(16 * g, 16)] = oy
                sbuf[16 + r, pl.ds(16 * g, 16)] = oz
                sbuf[24 + r, pl.ds(16 * g, 16)] = mv
            return 0

        lax.fori_loop(0, NG, per_group, 0)

    def compute_strip(s, sbuf):
        _compute(s, [sbuf], 8)

    def compute_strip2(s, bufP, bufQ):
        _compute(s, [bufP, bufQ], 16)

    def fire_strip(s, sbuf, sem):
        gn0 = pl.multiple_of(voff + 8 * s, 8)

        def row(q):
            return sbuf.at[pl.ds(8 * q, 8), pl.ds(0, BPH)]

        def scr(q):
            return scrT_hbm.at[pl.ds(0, 8), pl.ds(boff, BPH)]

        def out3(dst_hbm, loc, last):
            loc = pl.multiple_of(loc, 8)
            for c in range(3):
                pltpu.async_copy(
                    row(c), dst_hbm.at[c, pl.ds(loc, 8), pl.ds(boff, BPH)],
                    sem)
            pltpu.async_copy(row(3), last, sem)

        def v_case():
            out3(vT_hbm, gn0,
                 mT_hbm.at[pl.ds(pl.multiple_of(gn0, 8), 8), pl.ds(boff, BPH)])

        def rest1():
            lax.cond(gn0 < MAXLEN + NSUB,
                     lambda: out3(sT_hbm, gn0 - MAXLEN, scr(3)), rest2)

        def rest2():
            lax.cond(gn0 < MAXLEN + NSUB + 2 * NBBOX,
                     lambda: out3(bT_hbm, gn0 - (MAXLEN + NSUB), scr(3)),
                     rest3)

        def rest3():
            def k_case():
                out3(kT_hbm, gn0 - (MAXLEN + NSUB + 2 * NBBOX), scr(3))

            def pad_case():
                for q in range(4):
                    pltpu.async_copy(row(q), scr(q), sem)

            lax.cond(gn0 < NTOT, k_case, pad_case)

        lax.cond(gn0 < MAXLEN, v_case, rest1)

    def drain4(sbuf, sem):
        for q in range(4):
            pltpu.make_async_copy(
                scrT_hbm.at[pl.ds(0, 8), pl.ds(boff, BPH)],
                sbuf.at[pl.ds(8 * q, 8), pl.ds(0, BPH)], sem).wait()

    def quad_body(q, _):
        @pl.when(q > 0)
        def _():
            drain4(sbufA_v, semA)
            drain4(sbufB_v, semB)
        compute_strip2(4 * q, sbufA_v, sbufB_v)
        fire_strip(4 * q, sbufA_v, semA)
        fire_strip(4 * q + 1, sbufB_v, semB)

        @pl.when(q > 0)
        def _():
            drain4(sbufC_v, semC)
            drain4(sbufD_v, semD)
        compute_strip2(4 * q + 2, sbufC_v, sbufD_v)
        fire_strip(4 * q + 2, sbufC_v, semC)
        fire_strip(4 * q + 3, sbufD_v, semD)
        return 0

    lax.fori_loop(0, (NSTRIP - 1) // 4, quad_body, 0)
    drain4(sbufA_v, semA)
    drain4(sbufB_v, semB)
    compute_strip(NSTRIP - 1, sbufA_v)
    fire_strip(NSTRIP - 1, sbufA_v, semA)
    drain4(sbufC_v, semC)
    drain4(sbufD_v, semD)
    drain4(sbufA_v, semA)


_sc_kernel = pl.kernel(
    _sc_body,
    out_type=(
        jax.ShapeDtypeStruct((3, MAXLEN, B), jnp.float32),
        jax.ShapeDtypeStruct((3, NSUB, B), jnp.float32),
        jax.ShapeDtypeStruct((3, 2 * NBBOX, B), jnp.float32),
        jax.ShapeDtypeStruct((3, 2 * NKP, B), jnp.float32),
        jax.ShapeDtypeStruct((MAXLEN, B), jnp.float32),
        jax.ShapeDtypeStruct((B,), jnp.float32),
        jax.ShapeDtypeStruct((8, B), jnp.float32),
    ),
    mesh=plsc.VectorSubcoreMesh(core_axis_name="c", subcore_axis_name="s"),
    compiler_params=pltpu.CompilerParams(needs_layout_passes=False),
    scratch_types=[
        pltpu.VMEM((K * NPLANE * VPB,), jnp.float32),
        pltpu.VMEM((16 * BPH,), jnp.float32),
        pltpu.VMEM((BPH,), jnp.int32),
        pltpu.VMEM((16,), jnp.float32),
        pltpu.VMEM((BPH,), jnp.float32),
        pltpu.VMEM((32, BPH), jnp.float32),
        pltpu.VMEM((32, BPH), jnp.float32),
        pltpu.VMEM((32, BPH), jnp.float32),
        pltpu.VMEM((32, BPH), jnp.float32),
        pltpu.SemaphoreType.DMA,
        pltpu.SemaphoreType.DMA,
        pltpu.SemaphoreType.DMA,
        pltpu.SemaphoreType.DMA,
        pltpu.SemaphoreType.DMA,
    ],
)


def kernel(angles, global_orient, transl, obj_idx, v_tab, v_sub_tab, parts_tab,
           parts_sub_tab, bbox_top_tab, bbox_bottom_tab, kp_top_tab,
           kp_bottom_tab, mask_tab, diameter_tab):
    obj_idx = obj_idx.astype(jnp.int32)

    coords = jnp.concatenate(
        [v_tab, v_sub_tab, bbox_top_tab, bbox_bottom_tab, kp_top_tab,
         kp_bottom_tab], axis=1)
    flags = jnp.concatenate(
        [(parts_tab == 1).astype(jnp.float32),
         (parts_sub_tab == 1).astype(jnp.float32),
         jnp.ones((K, NBBOX), jnp.float32),
         jnp.zeros((K, NBBOX), jnp.float32),
         jnp.ones((K, NKP), jnp.float32),
         jnp.zeros((K, NKP), jnp.float32)], axis=1)
    maskp = jnp.pad(mask_tab.astype(jnp.float32), ((0, 0), (0, NTOT - MAXLEN)))
    planar = jnp.concatenate(
        [jnp.transpose(coords, (0, 2, 1)), flags[:, None, :],
         maskp[:, None, :]], axis=1)
    tabf = jnp.pad(planar, ((0, 0), (0, 0), (0, NPAD - NTOT))).reshape(-1)

    diam_pad = jnp.pad(diameter_tab.astype(jnp.float32), (0, 16 - K))
    coefT = _compute_coefs(angles, global_orient, transl)

    vT, sT, bT, kT, mT, diameter, _scr = _sc_kernel(
        tabf, coefT, obj_idx, diam_pad)
    return (jnp.transpose(vT, (2, 1, 0)), jnp.transpose(sT, (2, 1, 0)),
            jnp.transpose(bT, (2, 1, 0)), jnp.transpose(kT, (2, 1, 0)),
            diameter, jnp.transpose(mT, (1, 0)))


# speedup vs baseline: 1.5078x; 1.4119x over previous
import jax
import jax.numpy as jnp
from jax import lax
from jax.experimental import pallas as pl
from jax.experimental.pallas import tpu as pltpu
from jax.experimental.pallas import tpu_sc as plsc

K = 11
MAXLEN = 4000
NSUB = 600
NBBOX = 8
NKP = 16
B = 1024
NTOT = MAXLEN + NSUB + 2 * NBBOX + 2 * NKP
VB = 16
NPAD = 4736
VPB = NPAD // VB
NPLANE = 5
BPH = B // 2
NSTRIP = VPB // 8
NG = BPH // 16
KP = 16
TPW = VPB * NPLANE * KP


def _coef_body(ang_ref, gx_ref, gy_ref, gz_ref, tx_ref, ty_ref, tz_ref, out_ref):
    a = ang_ref[...]
    gx = gx_ref[...]
    gy = gy_ref[...]
    gz = gz_ref[...]

    n2 = gx * gx + gy * gy + gz * gz
    n = jnp.sqrt(n2)
    half = 0.5 * n
    small = n < 1e-6
    safe = jnp.where(small, 1.0, n)
    kf = jnp.where(small, 0.5 - n2 / 48.0, jnp.sin(half) / safe)
    w = jnp.cos(half)
    x = gx * kf
    y = gy * kf
    z = gz * kf
    wu = w * w - (x * x + y * y + z * z)
    out_ref[0] = wu + 2.0 * x * x
    out_ref[1] = 2.0 * (x * y - w * z)
    out_ref[2] = 2.0 * (x * z + w * y)
    out_ref[3] = 2.0 * (x * y + w * z)
    out_ref[4] = wu + 2.0 * y * y
    out_ref[5] = 2.0 * (y * z - w * x)
    out_ref[6] = 2.0 * (x * z - w * y)
    out_ref[7] = 2.0 * (y * z + w * x)
    out_ref[8] = wu + 2.0 * z * z

    an = jnp.abs(a)
    ah = 0.5 * an
    asmall = an < 1e-6
    asafe = jnp.where(asmall, 1.0, an)
    ak = jnp.where(asmall, 0.5 - an * an / 48.0, jnp.sin(ah) / asafe)
    aw = jnp.cos(ah)
    az = -a * ak
    out_ref[9] = aw * aw - az * az
    out_ref[10] = 2.0 * aw * az
    out_ref[11] = aw * aw + az * az
    out_ref[12] = tx_ref[...]
    out_ref[13] = ty_ref[...]
    out_ref[14] = tz_ref[...]
    out_ref[15] = jnp.zeros_like(a)


def _compute_coefs(angles, global_orient, transl):
    ang = angles.reshape(8, 128)
    g = global_orient.T.reshape(3, 8, 128)
    t = transl.T.reshape(3, 8, 128)
    planes = pl.pallas_call(
        _coef_body,
        out_shape=jax.ShapeDtypeStruct((16, 8, 128), jnp.float32),
    )(ang, g[0], g[1], g[2], t[0], t[1], t[2])
    return planes.reshape(16 * B)


def _sc_body(tabf_hbm, cT_hbm, idx_hbm, diam_hbm,
             vT_hbm, sT_hbm, bT_hbm, kT_hbm, mT_hbm, dout_hbm, scrT_hbm,
             tab_v, ctab_v, ktab_v, dtab_v, dbuf_v, sbufA_v, sbufB_v,
             sbufC_v, sbufD_v, sem, semA, semB, semC, semD):
    vb = lax.axis_index("s")
    bh = lax.axis_index("c")
    voff = pl.multiple_of(vb * VPB, 8)
    boff = pl.multiple_of(bh * BPH, BPH)

    copies = []
    copies.append(pltpu.async_copy(
        tabf_hbm.at[pl.ds(pl.multiple_of(vb * TPW, 8), TPW)], tab_v, sem))
    for j in range(16):
        copies.append(pltpu.async_copy(
            cT_hbm.at[pl.ds(pl.multiple_of(j * B + boff, 8), BPH)],
            ctab_v.at[pl.ds(j * BPH, BPH)], sem))
    copies.append(pltpu.async_copy(idx_hbm.at[pl.ds(boff, BPH)], ktab_v, sem))
    copies.append(pltpu.async_copy(diam_hbm, dtab_v, sem))
    for cp in copies:
        cp.wait()

    @pl.when(vb == 0)
    def _():
        def diam_g(g, _):
            kvec = ktab_v[pl.ds(16 * g, 16)]
            dbuf_v[pl.ds(16 * g, 16)] = plsc.load_gather(dtab_v, [kvec])
            return 0
        lax.fori_loop(0, NG, diam_g, 0)
        pltpu.sync_copy(dbuf_v, dout_hbm.at[pl.ds(boff, BPH)])

    def _compute(s, bufs, nrows):
        def per_group(g, _):
            kvec = ktab_v[pl.ds(16 * g, 16)]
            cv = [ctab_v[pl.ds(j * BPH + 16 * g, 16)] for j in range(15)]
            (m00, m01, m02, m10, m11, m12, m20, m21, m22,
             cC, cS, cZ, tx, ty, tz) = cv
            for nl in range(nrows):
                sbuf = bufs[nl // 8]
                r = nl % 8
                base = (8 * s + nl) * (NPLANE * KP)
                xv = plsc.load_gather(tab_v, [kvec + base])
                yv = plsc.load_gather(tab_v, [kvec + (base + KP)])
                zv = plsc.load_gather(tab_v, [kvec + (base + 2 * KP)])
                fv = plsc.load_gather(tab_v, [kvec + (base + 3 * KP)])
                mv = plsc.load_gather(tab_v, [kvec + (base + 4 * KP)])
                m = fv > 0.5
                xe = jnp.where(m, cC * xv - cS * yv, xv)
                ye = jnp.where(m, cS * xv + cC * yv, yv)
                ze = jnp.where(m, cZ * zv, zv)
                ox = m00 * xe + m01 * ye + m02 * ze + tx
                oy = m10 * xe + m11 * ye + m12 * ze + ty
                oz = m20 * xe + m21 * ye + m22 * ze + tz
                sbuf[r, pl.ds(16 * g, 16)] = ox
                sbuf[8 + r, pl.ds(16 * g, 16)] = oy
                sbuf[16 + r, pl.ds(16 * g, 16)] = oz
                sbuf[24 + r, pl.ds(16 * g, 16)] = mv
            return 0

        lax.fori_loop(0, NG, per_group, 0)

    def compute_strip(s, sbuf):
        _compute(s, [sbuf], 8)

    def compute_strip2(s, bufP, bufQ):
        _compute(s, [bufP, bufQ], 16)

    def fire_strip(s, sbuf, sem):
        gn0 = pl.multiple_of(voff + 8 * s, 8)

        def row(q):
            return sbuf.at[pl.ds(8 * q, 8), pl.ds(0, BPH)]

        def scr(q):
            return scrT_hbm.at[pl.ds(0, 8), pl.ds(boff, BPH)]

        def out3(dst_hbm, loc, last):
            loc = pl.multiple_of(loc, 8)
            for c in range(3):
                pltpu.async_copy(
                    row(c), dst_hbm.at[c, pl.ds(loc, 8), pl.ds(boff, BPH)],
                    sem)
            pltpu.async_copy(row(3), last, sem)

        def v_case():
            out3(vT_hbm, gn0,
                 mT_hbm.at[pl.ds(pl.multiple_of(gn0, 8), 8), pl.ds(boff, BPH)])

        def rest1():
            lax.cond(gn0 < MAXLEN + NSUB,
                     lambda: out3(sT_hbm, gn0 - MAXLEN, scr(3)), rest2)

        def rest2():
            lax.cond(gn0 < MAXLEN + NSUB + 2 * NBBOX,
                     lambda: out3(bT_hbm, gn0 - (MAXLEN + NSUB), scr(3)),
                     rest3)

        def rest3():
            def k_case():
                out3(kT_hbm, gn0 - (MAXLEN + NSUB + 2 * NBBOX), scr(3))

            def pad_case():
                for q in range(4):
                    pltpu.async_copy(row(q), scr(q), sem)

            lax.cond(gn0 < NTOT, k_case, pad_case)

        lax.cond(gn0 < MAXLEN, v_case, rest1)

    def drain4(sbuf, sem):
        for q in range(4):
            pltpu.make_async_copy(
                scrT_hbm.at[pl.ds(0, 8), pl.ds(boff, BPH)],
                sbuf.at[pl.ds(8 * q, 8), pl.ds(0, BPH)], sem).wait()

    def quad_body(q, _):
        @pl.when(q > 0)
        def _():
            drain4(sbufA_v, semA)
            drain4(sbufB_v, semB)
        compute_strip2(4 * q, sbufA_v, sbufB_v)
        fire_strip(4 * q, sbufA_v, semA)
        fire_strip(4 * q + 1, sbufB_v, semB)

        @pl.when(q > 0)
        def _():
            drain4(sbufC_v, semC)
            drain4(sbufD_v, semD)
        compute_strip2(4 * q + 2, sbufC_v, sbufD_v)
        fire_strip(4 * q + 2, sbufC_v, semC)
        fire_strip(4 * q + 3, sbufD_v, semD)
        return 0

    lax.fori_loop(0, (NSTRIP - 1) // 4, quad_body, 0)
    drain4(sbufA_v, semA)
    drain4(sbufB_v, semB)
    compute_strip(NSTRIP - 1, sbufA_v)
    fire_strip(NSTRIP - 1, sbufA_v, semA)
    drain4(sbufC_v, semC)
    drain4(sbufD_v, semD)
    drain4(sbufA_v, semA)


_sc_kernel = pl.kernel(
    _sc_body,
    out_type=(
        jax.ShapeDtypeStruct((3, MAXLEN, B), jnp.float32),
        jax.ShapeDtypeStruct((3, NSUB, B), jnp.float32),
        jax.ShapeDtypeStruct((3, 2 * NBBOX, B), jnp.float32),
        jax.ShapeDtypeStruct((3, 2 * NKP, B), jnp.float32),
        jax.ShapeDtypeStruct((MAXLEN, B), jnp.float32),
        jax.ShapeDtypeStruct((B,), jnp.float32),
        jax.ShapeDtypeStruct((8, B), jnp.float32),
    ),
    mesh=plsc.VectorSubcoreMesh(core_axis_name="c", subcore_axis_name="s"),
    compiler_params=pltpu.CompilerParams(needs_layout_passes=False),
    scratch_types=[
        pltpu.VMEM((TPW,), jnp.float32),
        pltpu.VMEM((16 * BPH,), jnp.float32),
        pltpu.VMEM((BPH,), jnp.int32),
        pltpu.VMEM((16,), jnp.float32),
        pltpu.VMEM((BPH,), jnp.float32),
        pltpu.VMEM((32, BPH), jnp.float32),
        pltpu.VMEM((32, BPH), jnp.float32),
        pltpu.VMEM((32, BPH), jnp.float32),
        pltpu.VMEM((32, BPH), jnp.float32),
        pltpu.SemaphoreType.DMA,
        pltpu.SemaphoreType.DMA,
        pltpu.SemaphoreType.DMA,
        pltpu.SemaphoreType.DMA,
        pltpu.SemaphoreType.DMA,
    ],
)


def kernel(angles, global_orient, transl, obj_idx, v_tab, v_sub_tab, parts_tab,
           parts_sub_tab, bbox_top_tab, bbox_bottom_tab, kp_top_tab,
           kp_bottom_tab, mask_tab, diameter_tab):
    obj_idx = obj_idx.astype(jnp.int32)

    coords = jnp.concatenate(
        [v_tab, v_sub_tab, bbox_top_tab, bbox_bottom_tab, kp_top_tab,
         kp_bottom_tab], axis=1)
    flags = jnp.concatenate(
        [(parts_tab == 1).astype(jnp.float32),
         (parts_sub_tab == 1).astype(jnp.float32),
         jnp.ones((K, NBBOX), jnp.float32),
         jnp.zeros((K, NBBOX), jnp.float32),
         jnp.ones((K, NKP), jnp.float32),
         jnp.zeros((K, NKP), jnp.float32)], axis=1)
    maskp = jnp.pad(mask_tab.astype(jnp.float32), ((0, 0), (0, NTOT - MAXLEN)))
    planar = jnp.concatenate(
        [jnp.transpose(coords, (0, 2, 1)), flags[:, None, :],
         maskp[:, None, :]], axis=1)
    planar = jnp.pad(planar, ((0, KP - K), (0, 0), (0, NPAD - NTOT)))
    tabf = jnp.transpose(planar, (2, 1, 0)).reshape(-1)

    diam_pad = jnp.pad(diameter_tab.astype(jnp.float32), (0, 16 - K))
    coefT = _compute_coefs(angles, global_orient, transl)

    vT, sT, bT, kT, mT, diameter, _scr = _sc_kernel(
        tabf, coefT, obj_idx, diam_pad)
    return (jnp.transpose(vT, (2, 1, 0)), jnp.transpose(sT, (2, 1, 0)),
            jnp.transpose(bT, (2, 1, 0)), jnp.transpose(kT, (2, 1, 0)),
            diameter, jnp.transpose(mT, (1, 0)))
